# baseline (speedup 1.0000x reference)
import jax
import jax.numpy as jnp
from jax import lax
from jax.experimental import pallas as pl
from jax.experimental.pallas import tpu as pltpu
from jax.experimental.pallas import tpu_sc as plsc

_B, _S, _D = 4096, 200, 64
_TOTAL = _B * _S
_NC, _NS = 2, 16
_NW = _NC * _NS
_BLK = _B // _NW
_PER_W = _BLK * _S
_HALF = _S // 2
_CHUNK = _BLK * _D


def _gather(emb_hbm, idx_t, rows_b, t, sem):
    pltpu.async_copy(emb_hbm.at[idx_t.at[pl.ds(t * _BLK, _BLK)]], rows_b, sem)


def _gather_wait(emb_hbm, rows_b, sem):
    pltpu.make_async_copy(emb_hbm.at[pl.ds(0, _BLK)], rows_b, sem).wait()


def _store(out_hbm, tbuf_b, t, w, sem):
    for dt in range(8):
        pltpu.async_copy(tbuf_b.at[pl.ds(dt * 1024, 1024)],
                         out_hbm.at[t, dt * 32 + w], sem)


def _store_wait(out_hbm, tbuf_b, w, sem):
    for dt in range(8):
        pltpu.make_async_copy(tbuf_b.at[pl.ds(dt * 1024, 1024)],
                              out_hbm.at[0, dt * 32 + w], sem).wait()


def _flatten_rows(rows_b, flat_b):
    def per_r(r2, carry):
        for rr in range(2):
            r = r2 * 2 + rr
            for j in range(4):
                flat_b[pl.ds(r * 64 + j * 16, 16)] = rows_b[r, pl.ds(j * 16,
                                                                     16)]
        return carry

    lax.fori_loop(0, _BLK // 2, per_r, 0)


def _transpose_rows(rows_b, tbuf_b, cvecs):
    def per_d(d, carry):
        dst = ((d >> 3) * 1024 + (d & 7) * 128).astype(jnp.int32)
        for c0 in range(8):
            v = plsc.load_gather(rows_b, [cvecs[c0] + d])
            tbuf_b[pl.ds(dst + c0 * 16, 16)] = v
        return carry

    lax.fori_loop(0, _D, per_d, 0)


def _emb_body(idx_hbm, emb_hbm, out_hbm,
              idx_raw, idx_t, rows0, rows1, flat0, flat1, tbuf0, tbuf1,
              g0, g1, s0, s1):
    w = lax.axis_index("s") * _NC + lax.axis_index("c")
    pltpu.sync_copy(idx_hbm.at[pl.ds(w * _PER_W, _PER_W)], idx_raw)

    iota16 = lax.iota(jnp.int32, 16)
    iota_s = iota16 * _S

    def tr_idx(t, carry):
        for b0 in range(8):
            src = iota_s + (b0 * 16 * _S + t)
            idx_t[pl.ds(t * _BLK + b0 * 16, 16)] = plsc.load_gather(
                idx_raw, [src])
        return carry

    lax.fori_loop(0, _S, tr_idx, 0)

    cvecs = [(iota16 + c0 * 16) * _D for c0 in range(8)]
    rows = (rows0, rows1)
    flat = (flat0, flat1)
    tbuf = (tbuf0, tbuf1)
    gsem = (g0, g1)
    ssem = (s0, s1)

    _gather(emb_hbm, idx_t, rows0, 0, g0)
    _gather(emb_hbm, idx_t, rows1, 1, g1)

    def body(u, carry):
        for b in range(2):
            t = 2 * u + b
            _gather_wait(emb_hbm, rows[b], gsem[b])

            @pl.when(u >= 1)
            def _():
                _store_wait(out_hbm, tbuf[b], w, ssem[b])

            _flatten_rows(rows[b], flat[b])
            _transpose_rows(flat[b], tbuf[b], cvecs)

            @pl.when(t + 2 < _S)
            def _():
                _gather(emb_hbm, idx_t, rows[b], t + 2, gsem[b])

            _store(out_hbm, tbuf[b], t, w, ssem[b])
        return carry

    lax.fori_loop(0, _HALF, body, 0)
    _store_wait(out_hbm, tbuf0, w, s0)
    _store_wait(out_hbm, tbuf1, w, s1)


def kernel(x, emb):
    idx = x.astype(jnp.int32).reshape(_TOTAL)
    emb_i = lax.bitcast_convert_type(emb, jnp.int32)
    run = pl.kernel(
        _emb_body,
        out_type=jax.ShapeDtypeStruct((_S, 256, 1024), jnp.int32),
        mesh=plsc.VectorSubcoreMesh(core_axis_name="c", subcore_axis_name="s"),
        compiler_params=pltpu.CompilerParams(use_tc_tiling_on_sc=False,
                                             needs_layout_passes=False),
        scratch_types=[
            pltpu.VMEM((_PER_W,), jnp.int32),
            pltpu.VMEM((_PER_W,), jnp.int32),
            pltpu.VMEM((_BLK, _D), jnp.int32),
            pltpu.VMEM((_BLK, _D), jnp.int32),
            pltpu.VMEM((_CHUNK,), jnp.int32),
            pltpu.VMEM((_CHUNK,), jnp.int32),
            pltpu.VMEM((_CHUNK,), jnp.int32),
            pltpu.VMEM((_CHUNK,), jnp.int32),
            pltpu.SemaphoreType.DMA,
            pltpu.SemaphoreType.DMA,
            pltpu.SemaphoreType.DMA,
            pltpu.SemaphoreType.DMA,
        ],
    )
    out = run(idx, emb_i)
    out = lax.bitcast_convert_type(out, jnp.float32)
    return (out.reshape(_S, 8, _NW, 8, 128)
               .transpose(2, 4, 0, 1, 3)
               .reshape(_B, _S, _D))


# speedup vs baseline: 2.0481x; 2.0481x over previous
import jax
import jax.numpy as jnp
from jax import lax
from jax.experimental import pallas as pl
from jax.experimental.pallas import tpu as pltpu
from jax.experimental.pallas import tpu_sc as plsc

_B, _S, _D = 4096, 200, 64
_TOTAL = _B * _S
_NC, _NS = 2, 16
_NW = _NC * _NS
_PER_W = _TOTAL // _NW
_G = 128
_NG = _PER_W // _G
_NBUF = 8
_T = _NG // _NBUF


def _gather(emb_hbm, idx_v, rows_v, j, b, sem):
    pltpu.async_copy(emb_hbm.at[idx_v.at[pl.ds(j * _G, _G)]], rows_v.at[b],
                     sem)


def _gather_wait(emb_hbm, rows_v, b, sem):
    pltpu.make_async_copy(emb_hbm.at[pl.ds(0, _G)], rows_v.at[b], sem).wait()


def _store(out_hbm, rows_v, base, j, b, sem):
    pltpu.async_copy(rows_v.at[b], out_hbm.at[base + j], sem)


def _store_wait(out_hbm, rows_v, b, sem):
    pltpu.make_async_copy(rows_v.at[b], out_hbm.at[0], sem).wait()


def _emb_body(idx_hbm, emb_hbm, out_hbm, idx_v, rows_v, *sems):
    gsems, ssems = sems[:_NBUF], sems[_NBUF:]
    wid = lax.axis_index("s") * _NC + lax.axis_index("c")
    pltpu.sync_copy(idx_hbm.at[pl.ds(wid * _PER_W, _PER_W)], idx_v)
    base = wid * _NG

    for b in range(_NBUF):
        _gather(emb_hbm, idx_v, rows_v, b, b, gsems[b])

    def body(t, carry):
        j0 = t * _NBUF
        for b in range(_NBUF):
            _gather_wait(emb_hbm, rows_v, b, gsems[b])
            _store(out_hbm, rows_v, base, j0 + b, b, ssems[b])

        @pl.when(t + 1 < _T)
        def _():
            for b in range(_NBUF):
                _store_wait(out_hbm, rows_v, b, ssems[b])
                _gather(emb_hbm, idx_v, rows_v, j0 + _NBUF + b, b, gsems[b])

        return carry

    lax.fori_loop(0, _T, body, 0)
    for b in range(_NBUF):
        _store_wait(out_hbm, rows_v, b, ssems[b])


def kernel(x, emb):
    idx = x.astype(jnp.int32).reshape(_TOTAL)
    run = pl.kernel(
        _emb_body,
        out_type=jax.ShapeDtypeStruct((_TOTAL // _G, _G, _D), jnp.float32),
        mesh=plsc.VectorSubcoreMesh(core_axis_name="c", subcore_axis_name="s"),
        compiler_params=pltpu.CompilerParams(use_tc_tiling_on_sc=False),
        scratch_types=(
            [pltpu.VMEM((_PER_W,), jnp.int32),
             pltpu.VMEM((_NBUF, _G, _D), jnp.float32)]
            + [pltpu.SemaphoreType.DMA] * (2 * _NBUF)
        ),
    )
    out = run(idx, emb)
    return out.reshape(_B, _S, _D)
